# Initial kernel scaffold; baseline (speedup 1.0000x reference)
import jax
import jax.numpy as jnp
from jax.experimental import pallas as pl


def kernel(input, weight, weight_scale):
    raise NotImplementedError("write your pallas kernel here")


# speedup vs baseline: 4.2681x; 4.2681x over previous
import functools

import jax
import jax.numpy as jnp
from jax import lax
from jax.experimental import pallas as pl
from jax.experimental.pallas import tpu as pltpu
from jax.experimental.pallas import tpu_sc as plsc

DIM = 64
WORDS = DIM // 4
L = 16
CH = 256
IDX_ROWS = CH // 128


def _qembed_body(n_chunks, nc, idx_hbm, wtab_hbm, stab_hbm, out_hbm,
                 idx_v, eidx_v, gidx_v, wrows_v, srows_v, orows_v, sem):
    wid = lax.axis_index("s") * nc + lax.axis_index("c")
    base0 = wid * (n_chunks * CH)

    def chunk_body(ci, carry):
        base = base0 + ci * CH
        pltpu.sync_copy(idx_hbm.at[pl.ds(base, CH)], idx_v)

        for t in range(CH // L):
            v = idx_v[pl.ds(t * L, L)]
            r, col = divmod(t * L, 128)
            eidx_v[r, pl.ds(col, L)] = v >> 1
            gidx_v[r, pl.ds(col, L)] = v >> 5

        copies = []
        for j in range(IDX_ROWS):
            copies.append(pltpu.async_copy(
                wtab_hbm.at[eidx_v.at[j]],
                wrows_v.at[pl.ds(j * 128, 128)], sem))
            copies.append(pltpu.async_copy(
                stab_hbm.at[gidx_v.at[j]],
                srows_v.at[pl.ds(j * 128, 128)], sem))
        for cp in copies:
            cp.wait()

        lane = lax.iota(jnp.int32, L)
        byte_shift = (lane & 3) << 3
        spread = lane >> 2

        def g_body(g, inner):
            r0 = g * L
            pshift = (idx_v[pl.ds(r0, L)] & 1) << 2
            for r in range(L):
                w = wrows_v[r0 + r]
                rsel = jnp.full((L,), r, jnp.int32)
                tshift = byte_shift + pshift.at[rsel].get(
                    mode="promise_in_bounds")
                for v in range(4):
                    shuf = w.at[spread + 4 * v].get(mode="promise_in_bounds")
                    nib = (shuf >> tshift) & 15
                    f = (nib - 8).astype(jnp.float32)
                    sc = srows_v[r0 + r, pl.ds(v * L, L)]
                    orows_v[r0 + r, pl.ds(v * L, L)] = f * sc
            return inner

        lax.fori_loop(0, CH // L, g_body, 0)
        pltpu.sync_copy(orows_v, out_hbm.at[pl.ds(base, CH)])
        return carry

    lax.fori_loop(0, n_chunks, chunk_body, 0)


def kernel(input, weight, weight_scale):
    n = input.size
    idx_flat = input.reshape(n)
    wtab = lax.bitcast_convert_type(
        weight.reshape(weight.shape[0], WORDS, 4), jnp.int32)

    mesh = plsc.VectorSubcoreMesh(core_axis_name="c", subcore_axis_name="s")
    nw = mesh.num_cores * mesh.num_subcores
    assert n % (nw * CH) == 0
    n_chunks = n // (nw * CH)

    grid_kernel = pl.kernel(
        functools.partial(_qembed_body, n_chunks, mesh.num_cores),
        out_type=jax.ShapeDtypeStruct((n, DIM), jnp.float32),
        mesh=mesh,
        scratch_types=[
            pltpu.VMEM((CH,), jnp.int32),
            pltpu.VMEM((IDX_ROWS, 128), jnp.int32),
            pltpu.VMEM((IDX_ROWS, 128), jnp.int32),
            pltpu.VMEM((CH, WORDS), jnp.int32),
            pltpu.VMEM((CH, DIM), jnp.float32),
            pltpu.VMEM((CH, DIM), jnp.float32),
            pltpu.SemaphoreType.DMA,
        ],
        compiler_params=pltpu.CompilerParams(use_tc_tiling_on_sc=False),
    )
    out = grid_kernel(idx_flat, wtab, weight_scale)
    return out.reshape(*input.shape, DIM)
